# baseline (speedup 1.0000x reference)
import functools

import jax
import jax.numpy as jnp
from jax import lax
from jax.experimental import pallas as pl
from jax.experimental.pallas import tpu as pltpu
from jax.experimental.pallas import tpu_sc as plsc

B = 16384
D = 64


def _sc_gather(uid, m1id, m2id, user_emb, movie_emb):
    info = plsc.get_sparse_core_info()
    nc, ns = info.num_cores, info.num_subcores
    nw = nc * ns
    bpw = B // nw
    nchunk = bpw // 128

    mesh = plsc.VectorSubcoreMesh(core_axis_name="c", subcore_axis_name="s")

    @functools.partial(
        pl.kernel,
        mesh=mesh,
        compiler_params=pltpu.CompilerParams(use_tc_tiling_on_sc=False),
        out_type=[jax.ShapeDtypeStruct((B, D), jnp.float32)] * 3,
        scratch_types=[
            pltpu.VMEM((nchunk, 128), jnp.int32),
            pltpu.VMEM((nchunk, 128), jnp.int32),
            pltpu.VMEM((nchunk, 128), jnp.int32),
            pltpu.VMEM((bpw, D), jnp.float32),
            pltpu.VMEM((bpw, D), jnp.float32),
            pltpu.VMEM((bpw, D), jnp.float32),
            pltpu.SemaphoreType.DMA,
        ],
    )
    def gather_k(uid_hbm, m1_hbm, m2_hbm, uemb_hbm, memb_hbm,
                 out_u, out_1, out_2,
                 idx_u, idx_1, idx_2, rows_u, rows_1, rows_2, sem):
        wid = lax.axis_index("s") * nc + lax.axis_index("c")
        base = wid * bpw
        pltpu.sync_copy(uid_hbm.at[pl.ds(wid * nchunk, nchunk)], idx_u)
        pltpu.sync_copy(m1_hbm.at[pl.ds(wid * nchunk, nchunk)], idx_1)
        pltpu.sync_copy(m2_hbm.at[pl.ds(wid * nchunk, nchunk)], idx_2)
        copies = []
        for j in range(nchunk):
            dst = pl.ds(j * 128, 128)
            copies.append(
                pltpu.async_copy(uemb_hbm.at[idx_u.at[j]], rows_u.at[dst], sem))
            copies.append(
                pltpu.async_copy(memb_hbm.at[idx_1.at[j]], rows_1.at[dst], sem))
            copies.append(
                pltpu.async_copy(memb_hbm.at[idx_2.at[j]], rows_2.at[dst], sem))
        for c in copies:
            c.wait()
        pltpu.sync_copy(rows_u, out_u.at[pl.ds(base, bpw)])
        pltpu.sync_copy(rows_1, out_1.at[pl.ds(base, bpw)])
        pltpu.sync_copy(rows_2, out_2.at[pl.ds(base, bpw)])

    return gather_k(uid, m1id, m2id, user_emb, movie_emb)


def _mlp_body(u_ref, v1_ref, v2_ref, w1u_ref, w1m_ref, b1_ref, w2_ref,
              b2_ref, w3_ref, o_ref):
    uw = jnp.dot(u_ref[...], w1u_ref[...], preferred_element_type=jnp.float32)

    def head(v_ref):
        h = uw + jnp.dot(v_ref[...], w1m_ref[...],
                         preferred_element_type=jnp.float32) + b1_ref[...]
        h = jnp.maximum(h, 0.0)
        h = jnp.dot(h, w2_ref[...], preferred_element_type=jnp.float32)
        h = jnp.maximum(h + b2_ref[...], 0.0)
        return h

    dh = head(v1_ref) - head(v2_ref)
    o_ref[...] = jnp.sum(dh * w3_ref[...], axis=1, keepdims=True)


def _tc_mlp(u, v1, v2, w1u, w1m, b1, w2, b2, w3):
    blk = 2048
    grid = B // blk
    row = lambda i: (i, 0)
    const = lambda i: (0, 0)
    return pl.pallas_call(
        _mlp_body,
        grid=(grid,),
        in_specs=[
            pl.BlockSpec((blk, D), row),
            pl.BlockSpec((blk, D), row),
            pl.BlockSpec((blk, D), row),
            pl.BlockSpec((D, 16), const),
            pl.BlockSpec((D, 16), const),
            pl.BlockSpec((1, 16), const),
            pl.BlockSpec((16, 8), const),
            pl.BlockSpec((1, 8), const),
            pl.BlockSpec((1, 8), const),
        ],
        out_specs=pl.BlockSpec((blk, 1), row),
        out_shape=jax.ShapeDtypeStruct((B, 1), jnp.float32),
    )(u, v1, v2, w1u, w1m, b1, w2, b2, w3)


def kernel(inputs, user_emb, movie_emb, W1, b1, W2, b2, W3, b3):
    idx = inputs.astype(jnp.int32)
    uid = idx[:, 0].reshape(B // 128, 128)
    m1id = idx[:, 1].reshape(B // 128, 128)
    m2id = idx[:, 2].reshape(B // 128, 128)
    u, v1, v2 = _sc_gather(uid, m1id, m2id, user_emb, movie_emb)
    return _tc_mlp(
        u, v1, v2,
        W1[:D], W1[D:],
        b1.reshape(1, 16),
        W2,
        b2.reshape(1, 8),
        W3.reshape(1, 8),
    )
